# baseline (speedup 1.0000x reference)
import functools

import jax
import jax.numpy as jnp
from jax import lax
from jax.experimental import pallas as pl
from jax.experimental.pallas import tpu as pltpu
from jax.experimental.pallas import tpu_sc as plsc

_NUM_CORES = 2
_NUM_SUBCORES = 16
_NUM_WORKERS = _NUM_CORES * _NUM_SUBCORES
_CHUNK = 128


@functools.partial(jax.jit, static_argnames=("total", "dim"))
def _sc_gather(weight, flat_idx, *, total, dim):
    per_w = total // _NUM_WORKERS
    n_chunks = per_w // _CHUNK

    mesh = plsc.VectorSubcoreMesh(core_axis_name="c", subcore_axis_name="s")

    @functools.partial(
        pl.kernel,
        mesh=mesh,
        out_type=jax.ShapeDtypeStruct((total, dim), jnp.float32),
        scratch_types=[
            pltpu.VMEM((_CHUNK,), jnp.int32),
            pltpu.VMEM((_CHUNK, dim), jnp.float32),
            pltpu.SemaphoreType.DMA,
        ],
        compiler_params=pltpu.CompilerParams(use_tc_tiling_on_sc=False),
    )
    def k(table_hbm, idx_hbm, out_hbm, idx_v, rows_v, sem):
        wid = lax.axis_index("s") * _NUM_CORES + lax.axis_index("c")
        base = wid * per_w

        def step(j, carry):
            off = pl.multiple_of(base + j * _CHUNK, _CHUNK)
            pltpu.sync_copy(idx_hbm.at[pl.ds(off, _CHUNK)], idx_v)
            pltpu.async_copy(table_hbm.at[idx_v], rows_v, sem).wait()
            pltpu.sync_copy(rows_v, out_hbm.at[pl.ds(off, _CHUNK)])
            return carry

        lax.fori_loop(0, n_chunks, step, 0)

    return k(weight, flat_idx)


def kernel(x, weight):
    b, h = x.shape
    v, d = weight.shape
    total = b * h
    flat = x.reshape(total).astype(jnp.int32)
    out = _sc_gather(weight, flat, total=total, dim=d)
    return out.reshape(b, h, d)


# speedup vs baseline: 1.3064x; 1.3064x over previous
import functools

import jax
import jax.numpy as jnp
from jax import lax
from jax.experimental import pallas as pl
from jax.experimental.pallas import tpu as pltpu
from jax.experimental.pallas import tpu_sc as plsc

_NUM_CORES = 2
_NUM_SUBCORES = 16
_NUM_WORKERS = _NUM_CORES * _NUM_SUBCORES
_CHUNK = 128
_NB = 8
_AHEAD = 4
_SUPER = 200


@functools.partial(jax.jit, static_argnames=("total", "dim"))
def _sc_gather(weight, idx2d, *, total, dim):
    per_w = total // (_NUM_WORKERS * _CHUNK)
    n_super = per_w // _SUPER
    k_main = _SUPER // _NB

    mesh = plsc.VectorSubcoreMesh(core_axis_name="c", subcore_axis_name="s")

    @functools.partial(
        pl.kernel,
        mesh=mesh,
        out_type=jax.ShapeDtypeStruct((total, dim), jnp.float32),
        scratch_types=[
            pltpu.VMEM((2, _SUPER, _CHUNK), jnp.int32),
            pltpu.VMEM((_NB, _CHUNK, dim), jnp.float32),
        ] + [pltpu.SemaphoreType.DMA] * (2 * _NB + 2),
        compiler_params=pltpu.CompilerParams(use_tc_tiling_on_sc=False),
    )
    def k(table_hbm, idx_hbm, out_hbm, idx_v, rows_v, *sems):
        gsem = sems[:_NB]
        ssem = sems[_NB:2 * _NB]
        isem = sems[2 * _NB:]

        wid = lax.axis_index("s") * _NUM_CORES + lax.axis_index("c")
        base = wid * per_w

        def fire_idx(s, slot):
            pltpu.async_copy(
                idx_hbm.at[pl.ds(base + s * _SUPER, _SUPER)],
                idx_v.at[slot], isem[slot])

        def wait_idx(slot):
            pltpu.make_async_copy(
                idx_hbm.at[pl.ds(0, _SUPER)], idx_v.at[slot],
                isem[slot]).wait()

        def fire_gather(slot, c, b):
            pltpu.async_copy(
                table_hbm.at[idx_v.at[slot, c]], rows_v.at[b], gsem[b])

        def wait_gather(b):
            pltpu.make_async_copy(
                table_hbm.at[pl.ds(0, _CHUNK)], rows_v.at[b], gsem[b]).wait()

        def fire_store(c_glob, b):
            off = pl.multiple_of(c_glob * _CHUNK, _CHUNK)
            pltpu.async_copy(
                rows_v.at[b], out_hbm.at[pl.ds(off, _CHUNK)], ssem[b])

        def wait_store(b):
            pltpu.make_async_copy(
                rows_v.at[b], out_hbm.at[pl.ds(0, _CHUNK)], ssem[b]).wait()

        fire_idx(0, 0)
        for s in range(n_super):
            slot = s % 2
            wait_idx(slot)
            if s + 1 < n_super:
                fire_idx(s + 1, 1 - slot)
            sb = base + s * _SUPER

            for b in range(_AHEAD):
                fire_gather(slot, b, b)

            for j in range(_NB):
                wait_gather(j)
                fire_store(sb + j, j)
                c2 = j + _AHEAD
                b2 = c2 % _NB
                if c2 >= _NB:
                    wait_store(b2)
                fire_gather(slot, c2, b2)

            def body(kk, carry):
                for j in range(_NB):
                    c = kk * _NB + j
                    wait_gather(j)
                    fire_store(sb + c, j)
                    b2 = (j + _AHEAD) % _NB
                    wait_store(b2)
                    fire_gather(slot, c + _AHEAD, b2)
                return carry

            lax.fori_loop(1, k_main - 1, body, 0)

            for j in range(_NB):
                c = (k_main - 1) * _NB + j
                wait_gather(j)
                fire_store(sb + c, j)
                if j < _NB - _AHEAD:
                    b2 = (j + _AHEAD) % _NB
                    wait_store(b2)
                    fire_gather(slot, c + _AHEAD, b2)

            for b in range(_NB):
                wait_store(b)

    return k(weight, idx2d)


def kernel(x, weight):
    b, h = x.shape
    v, d = weight.shape
    total = b * h
    idx2d = x.reshape(total // _CHUNK, _CHUNK).astype(jnp.int32)
    out = _sc_gather(weight, idx2d, total=total, dim=d)
    return out.reshape(b, h, d)
